# baseline (speedup 1.0000x reference)
import functools

import jax
import jax.numpy as jnp
from jax import lax
from jax.experimental import pallas as pl
from jax.experimental.pallas import tpu as pltpu
from jax.experimental.pallas import tpu_sc as plsc

_K = 1000
_B = 1024
_S = 26
_NC, _NS, _L = 2, 16, 16
_NW = _NC * _NS
_KC = 40
_CPS = _K // _KC
_NCHUNK = _S * _CPS
_BASE = _NCHUNK // _NW
_EXTRA = _NCHUNK % _NW
_SLOTS = _BASE + 1
_GROUPS = _B // _L
_NBUF = 3


def _body(idx_hbm, out_hbm, idx_v, buf0, buf1, buf2, sem0, sem1, sem2):
    wid = lax.axis_index("s") * _NC + lax.axis_index("c")
    start = _BASE * wid + jnp.minimum(wid, _EXTRA)
    jl = jnp.minimum(start // _CPS, _S - 2)

    pltpu.sync_copy(idx_hbm.at[pl.ds(jl * _B, 2 * _B)], idx_v)

    zeros = jnp.zeros((_L,), jnp.int32)
    ones = jnp.full((_L,), 1, jnp.int32)
    iota = lax.iota(jnp.int32, _L)

    def _chunk_coords(t):
        c = start + t
        j = c // _CPS
        k0 = (c - j * _CPS) * _KC
        return j - jl, k0

    def _scan_scatter(buf, jr, k0, value):
        def _g(g, carry):
            vals = idx_v[pl.ds(jr * _B + g * _L, _L)]
            rows = vals - k0
            mask = (rows >= 0) & (rows < _KC)
            cols = g * _L + iota
            plsc.store_scatter(buf, [rows, cols], value, mask=mask)
            return carry

        lax.fori_loop(0, _GROUPS, _g, 0, unroll=4)

    def _zero(buf):
        def _z(r, carry):
            for o in range(_B // _L):
                buf[r, pl.ds(o * _L, _L)] = zeros
            return carry

        lax.fori_loop(0, _KC, _z, 0)

    bufs = (buf0, buf1, buf2)
    sems = (sem0, sem1, sem2)
    copies = [None] * _NBUF
    prev = [None] * _NBUF

    for t in range(_SLOTS):
        b = t % _NBUF
        buf = bufs[b]
        if t < _NBUF:
            _zero(buf)
        else:
            copies[b].wait()
            _scan_scatter(buf, prev[b][0], prev[b][1], zeros)
        jr, k0 = _chunk_coords(t)
        j = jr + jl
        if t < _BASE:
            _scan_scatter(buf, jr, k0, ones)
            dst = out_hbm.at[j, pl.ds(k0, _KC)]
            copies[b] = pltpu.async_copy(buf, dst, sems[b])
            prev[b] = (jr, k0)
        else:
            @pl.when(wid < _EXTRA)
            def _():
                _scan_scatter(buf, jr, k0, ones)
                dst = out_hbm.at[j, pl.ds(k0, _KC)]
                pltpu.async_copy(buf, dst, sems[b]).wait()

    for tp in range(_SLOTS - _NBUF, _BASE):
        copies[tp % _NBUF].wait()


_sc_onehot = functools.partial(
    pl.kernel,
    out_type=jax.ShapeDtypeStruct((_S, _K, _B), jnp.int32),
    mesh=plsc.VectorSubcoreMesh(core_axis_name="c", subcore_axis_name="s"),
    compiler_params=pltpu.CompilerParams(needs_layout_passes=False, skip_device_barrier=True),
    scratch_types=[
        pltpu.VMEM((2 * _B,), jnp.int32),
        pltpu.VMEM((_KC, _B), jnp.int32),
        pltpu.VMEM((_KC, _B), jnp.int32),
        pltpu.VMEM((_KC, _B), jnp.int32),
        pltpu.SemaphoreType.DMA,
        pltpu.SemaphoreType.DMA,
        pltpu.SemaphoreType.DMA,
    ],
)(_body)


@jax.jit
def kernel(tensor):
    idx_t = tensor.T.astype(jnp.int32).reshape(-1)
    o = _sc_onehot(idx_t)
    return jnp.transpose(o, (2, 0, 1))


# speedup vs baseline: 1.0387x; 1.0387x over previous
import functools

import jax
import jax.numpy as jnp
from jax import lax
from jax.experimental import pallas as pl
from jax.experimental.pallas import tpu as pltpu
from jax.experimental.pallas import tpu_sc as plsc

_K = 1000
_B = 1024
_S = 26
_NC, _NS, _L = 2, 16, 16
_NW = _NC * _NS
_KC = 40
_CPS = _K // _KC
_NCHUNK = _S * _CPS
_BASE = _NCHUNK // _NW
_EXTRA = _NCHUNK % _NW
_GROUPS = _B // _L
_NBUF = 3
_MAIN_END = _BASE - (_BASE - _NBUF) % _NBUF


def _body(idx_hbm, out_hbm, idx_v, buf0, buf1, buf2, sem0, sem1, sem2):
    wid = lax.axis_index("s") * _NC + lax.axis_index("c")
    start = _BASE * wid + jnp.minimum(wid, _EXTRA)
    jl = jnp.minimum(start // _CPS, _S - 2)

    pltpu.sync_copy(idx_hbm.at[pl.ds(jl * _B, 2 * _B)], idx_v)

    zeros = jnp.zeros((_L,), jnp.int32)
    ones = jnp.full((_L,), 1, jnp.int32)
    iota = lax.iota(jnp.int32, _L)
    bufs = (buf0, buf1, buf2)
    sems = (sem0, sem1, sem2)

    def _coords(t):
        c = start + t
        j = c // _CPS
        k0 = (c - j * _CPS) * _KC
        return j - jl, k0

    def _scan_scatter(buf, jr, k0, value):
        def _g(g, carry):
            vals = idx_v[pl.ds(jr * _B + g * _L, _L)]
            rows = vals - k0
            mask = (rows >= 0) & (rows < _KC)
            cols = g * _L + iota
            plsc.store_scatter(buf, [rows, cols], value, mask=mask)
            return carry

        lax.fori_loop(0, _GROUPS, _g, 0, unroll=2)

    def _emit(b, t):
        jr, k0 = _coords(t)
        _scan_scatter(bufs[b], jr, k0, ones)
        dst = out_hbm.at[jr + jl, pl.ds(k0, _KC)]
        return pltpu.async_copy(bufs[b], dst, sems[b])

    def _ring_wait(b):
        pltpu.make_async_copy(bufs[b], out_hbm.at[0, pl.ds(0, _KC)],
                              sems[b]).wait()

    def _zero(buf):
        def _z(r, carry):
            for o in range(_B // _L):
                buf[r, pl.ds(o * _L, _L)] = zeros
            return carry

        lax.fori_loop(0, _KC, _z, 0)

    for t in range(_NBUF):
        _zero(bufs[t])
        _emit(t, t)

    def _cycle(it, carry):
        t = _NBUF + it * _NBUF
        for b in range(_NBUF):
            _ring_wait(b)
            jr, k0 = _coords(t + b - _NBUF)
            _scan_scatter(bufs[b], jr, k0, zeros)
            _emit(b, t + b)
        return carry

    lax.fori_loop(0, (_MAIN_END - _NBUF) // _NBUF, _cycle, 0)

    for t in range(_MAIN_END, _BASE):
        b = t % _NBUF
        _ring_wait(b)
        jr, k0 = _coords(t - _NBUF)
        _scan_scatter(bufs[b], jr, k0, zeros)
        _emit(b, t)

    bl = _BASE % _NBUF
    _ring_wait(bl)

    @pl.when(wid < _EXTRA)
    def _():
        jr, k0 = _coords(_BASE - _NBUF)
        _scan_scatter(bufs[bl], jr, k0, zeros)
        _emit(bl, _BASE).wait()

    for t in range(_BASE - _NBUF + 1, _BASE):
        _ring_wait(t % _NBUF)


_sc_onehot = functools.partial(
    pl.kernel,
    out_type=jax.ShapeDtypeStruct((_S, _K, _B), jnp.int32),
    mesh=plsc.VectorSubcoreMesh(core_axis_name="c", subcore_axis_name="s"),
    compiler_params=pltpu.CompilerParams(needs_layout_passes=False),
    scratch_types=[
        pltpu.VMEM((2 * _B,), jnp.int32),
        pltpu.VMEM((_KC, _B), jnp.int32),
        pltpu.VMEM((_KC, _B), jnp.int32),
        pltpu.VMEM((_KC, _B), jnp.int32),
        pltpu.SemaphoreType.DMA,
        pltpu.SemaphoreType.DMA,
        pltpu.SemaphoreType.DMA,
    ],
)(_body)


@jax.jit
def kernel(tensor):
    idx_t = tensor.T.astype(jnp.int32).reshape(-1)
    o = _sc_onehot(idx_t)
    return jnp.transpose(o, (2, 0, 1))


# speedup vs baseline: 1.0663x; 1.0266x over previous
import functools

import jax
import jax.numpy as jnp
from jax import lax
from jax.experimental import pallas as pl
from jax.experimental.pallas import tpu as pltpu
from jax.experimental.pallas import tpu_sc as plsc

_K = 1000
_B = 1024
_S = 26
_NC, _NS, _L = 2, 16, 16
_NW = _NC * _NS
_KC = 40
_CPS = _K // _KC
_NCHUNK = _S * _CPS
_BASE = _NCHUNK // _NW
_EXTRA = _NCHUNK % _NW
_GROUPS = _B // _L
_NBUF = 3
_MAIN_END = _BASE - (_BASE - _NBUF) % _NBUF


def _body(idx_hbm, out_hbm, idx_v, buf0, buf1, buf2, sem0, sem1, sem2):
    wid = lax.axis_index("s") * _NC + lax.axis_index("c")
    start = _BASE * wid + jnp.minimum(wid, _EXTRA)
    jl = jnp.minimum(start // _CPS, _S - 2)

    zeros = jnp.zeros((_L,), jnp.int32)
    ones = jnp.full((_L,), 1, jnp.int32)
    iota = lax.iota(jnp.int32, _L)
    bufs = (buf0, buf1, buf2)
    sems = (sem0, sem1, sem2)

    def _coords(t):
        c = start + t
        j = c // _CPS
        k0 = (c - j * _CPS) * _KC
        return j - jl, k0

    def _scan_scatter(buf, jr, k0, value):
        def _g(g, carry):
            vals = idx_v[pl.ds(jr * _B + g * _L, _L)]
            rows = vals - k0
            mask = (rows >= 0) & (rows < _KC)
            cols = g * _L + iota
            plsc.store_scatter(buf, [rows, cols], value, mask=mask)
            return carry

        lax.fori_loop(0, _GROUPS, _g, 0, unroll=2)

    def _emit(b, t):
        jr, k0 = _coords(t)
        _scan_scatter(bufs[b], jr, k0, ones)
        dst = out_hbm.at[jr + jl, pl.ds(k0, _KC)]
        return pltpu.async_copy(bufs[b], dst, sems[b])

    def _ring_wait(b):
        pltpu.make_async_copy(bufs[b], out_hbm.at[0, pl.ds(0, _KC)],
                              sems[b]).wait()

    def _zero(buf):
        def _z(r, carry):
            for o in range(_B // _L):
                buf[r, pl.ds(o * _L, _L)] = zeros
            return carry

        lax.fori_loop(0, _KC, _z, 0)

    idx_cp = pltpu.async_copy(idx_hbm.at[pl.ds(jl * _B, 2 * _B)], idx_v,
                              sem0)
    _zero(bufs[0])
    idx_cp.wait()
    _emit(0, 0)
    for t in range(1, _NBUF):
        _zero(bufs[t])
        _emit(t, t)

    def _cycle(it, carry):
        t = _NBUF + it * _NBUF
        for b in range(_NBUF):
            _ring_wait(b)
            jr, k0 = _coords(t + b - _NBUF)
            _scan_scatter(bufs[b], jr, k0, zeros)
            _emit(b, t + b)
        return carry

    lax.fori_loop(0, (_MAIN_END - _NBUF) // _NBUF, _cycle, 0)

    for t in range(_MAIN_END, _BASE):
        b = t % _NBUF
        _ring_wait(b)
        jr, k0 = _coords(t - _NBUF)
        _scan_scatter(bufs[b], jr, k0, zeros)
        _emit(b, t)

    bl = _BASE % _NBUF
    _ring_wait(bl)

    @pl.when(wid < _EXTRA)
    def _():
        jr, k0 = _coords(_BASE - _NBUF)
        _scan_scatter(bufs[bl], jr, k0, zeros)
        _emit(bl, _BASE).wait()

    for t in range(_BASE - _NBUF + 1, _BASE):
        _ring_wait(t % _NBUF)


_sc_onehot = functools.partial(
    pl.kernel,
    out_type=jax.ShapeDtypeStruct((_S, _K, _B), jnp.int32),
    mesh=plsc.VectorSubcoreMesh(core_axis_name="c", subcore_axis_name="s"),
    compiler_params=pltpu.CompilerParams(needs_layout_passes=False),
    scratch_types=[
        pltpu.VMEM((2 * _B,), jnp.int32),
        pltpu.VMEM((_KC, _B), jnp.int32),
        pltpu.VMEM((_KC, _B), jnp.int32),
        pltpu.VMEM((_KC, _B), jnp.int32),
        pltpu.SemaphoreType.DMA,
        pltpu.SemaphoreType.DMA,
        pltpu.SemaphoreType.DMA,
    ],
)(_body)


@jax.jit
def kernel(tensor):
    idx_t = tensor.T.astype(jnp.int32).reshape(-1)
    o = _sc_onehot(idx_t)
    return jnp.transpose(o, (2, 0, 1))
